# baseline (speedup 1.0000x reference)
import functools
import math

import jax
import jax.numpy as jnp
from jax import lax
from jax.experimental import pallas as pl
from jax.experimental.pallas import tpu as pltpu
from jax.experimental.pallas import tpu_sc as plsc

D_MODEL = 64
_SCALE = math.sqrt(D_MODEL)
_BLK = 128
_NBUF = 4
_LOOK = 2


@functools.lru_cache(maxsize=None)
def _make_sc_kernel(seq_len: int, batch: int, vocab2: int):
    info = plsc.get_sparse_core_info()
    num_workers = info.num_cores * info.num_subcores
    assert batch == num_workers * _BLK
    n_chunks = seq_len
    assert n_chunks % _NBUF == 0 and n_chunks >= 2 * _NBUF

    mesh = plsc.VectorSubcoreMesh(core_axis_name="c", subcore_axis_name="s")

    @functools.partial(
        pl.kernel,
        mesh=mesh,
        out_type=jax.ShapeDtypeStruct((seq_len, D_MODEL, batch), jnp.float32),
        scratch_types=(
            [pltpu.VMEM((_BLK,), jnp.int32) for _ in range(2 * _NBUF)]
            + [pltpu.VMEM((_BLK,), jnp.int32) for _ in range(_NBUF)]
            + [pltpu.VMEM((_BLK, 128), jnp.float32) for _ in range(_NBUF)]
            + [pltpu.VMEM((D_MODEL, _BLK), jnp.float32) for _ in range(_NBUF)]
            + [pltpu.SemaphoreType.DMA for _ in range(2 * _NBUF)]
        ),
        compiler_params=pltpu.CompilerParams(
            use_tc_tiling_on_sc=True,
            needs_layout_passes=False,
            skip_device_barrier=True,
            disable_semaphore_checks=True,
            disable_bounds_checks=True,
        ),
    )
    def sc_kernel(xt_hbm, lut2_hbm, out_hbm, *scratch):
        idx2_bufs = scratch[:_NBUF]
        raw_bufs = scratch[_NBUF : 2 * _NBUF]
        col_bufs = scratch[2 * _NBUF : 3 * _NBUF]
        rows_bufs = scratch[3 * _NBUF : 4 * _NBUF]
        out_bufs = scratch[4 * _NBUF : 5 * _NBUF]
        gsems = scratch[5 * _NBUF : 6 * _NBUF]
        osems = scratch[6 * _NBUF : 7 * _NBUF]

        wid = lax.axis_index("s") * info.num_cores + lax.axis_index("c")
        b0 = wid * _BLK

        def issue_gather(s, b):
            pltpu.sync_copy(xt_hbm.at[s, pl.ds(b0, _BLK)], raw_bufs[b])
            for j in range(_BLK // 16):
                sl = pl.ds(j * 16, 16)
                v = raw_bufs[b][sl]
                idx2_bufs[b][sl] = jnp.right_shift(v, 1)
                col_bufs[b][sl] = jnp.left_shift(jnp.bitwise_and(v, 1), 6)
            pltpu.async_copy(lut2_hbm.at[idx2_bufs[b]], rows_bufs[b], gsems[b])

        def wait_gather(b):
            pltpu.make_async_copy(
                lut2_hbm.at[idx2_bufs[b]], rows_bufs[b], gsems[b]
            ).wait()

        def issue_out(s, b):
            pltpu.async_copy(
                out_bufs[b], out_hbm.at[s, :, pl.ds(b0, _BLK)], osems[b]
            )

        def wait_out(b):
            pltpu.make_async_copy(
                out_bufs[b], out_hbm.at[0, :, pl.ds(b0, _BLK)], osems[b]
            ).wait()

        def register_pass(b):
            iota = lax.iota(jnp.int32, 16)
            unroll = 8
            for j in range(_BLK // 16):
                tok = j * 16 + iota
                colv = col_bufs[b][pl.ds(j * 16, 16)]

                def dbody(dd, carry):
                    d0 = dd * unroll
                    for k in range(unroll):
                        d = d0 + k
                        vals = plsc.load_gather(rows_bufs[b], [tok, colv + d])
                        out_bufs[b][d, pl.ds(j * 16, 16)] = vals * _SCALE
                    return carry

                lax.fori_loop(0, D_MODEL // unroll, dbody, 0)

        for i in range(_LOOK):
            issue_gather(i, i)

        def outer(it, carry):
            for b in range(_NBUF):
                i = it * _NBUF + b
                j = i + _LOOK
                bj = (b + _LOOK) % _NBUF

                @pl.when(jnp.logical_and(j >= _NBUF, j < n_chunks))
                def _():
                    wait_out(bj)

                @pl.when(j < n_chunks)
                def _():
                    issue_gather(j, bj)

                wait_gather(b)
                register_pass(b)
                issue_out(i, b)
            return carry

        lax.fori_loop(0, n_chunks // _NBUF, outer, 0)

        for b in range(_NBUF):
            wait_out(b)

    return sc_kernel


def kernel(x, lut):
    batch, seq = x.shape
    vocab = lut.shape[0]
    xt = x.T
    lut2 = lut.reshape(vocab // 2, 2 * D_MODEL)
    out_t = _make_sc_kernel(seq, batch, vocab // 2)(xt, lut2)
    return out_t.transpose(2, 0, 1)


# speedup vs baseline: 1.1632x; 1.1632x over previous
import functools
import math

import jax
import jax.numpy as jnp
from jax import lax
from jax.experimental import pallas as pl
from jax.experimental.pallas import tpu as pltpu
from jax.experimental.pallas import tpu_sc as plsc

D_MODEL = 64
_SCALE = math.sqrt(D_MODEL)
_BLK = 128
_NBUF = 4
_LOOK = 2

_SC_PARAMS = pltpu.CompilerParams(
    use_tc_tiling_on_sc=True,
    needs_layout_passes=False,
    skip_device_barrier=True,
    disable_semaphore_checks=True,
    disable_bounds_checks=True,
)


@functools.lru_cache(maxsize=None)
def _make_gather_kernel(seq_len: int, batch: int, vocab: int):
    info = plsc.get_sparse_core_info()
    num_workers = info.num_cores * info.num_subcores
    assert batch == num_workers * _BLK
    n_chunks = seq_len
    assert n_chunks % _NBUF == 0 and n_chunks >= 2 * _NBUF

    mesh = plsc.VectorSubcoreMesh(core_axis_name="c", subcore_axis_name="s")

    @functools.partial(
        pl.kernel,
        mesh=mesh,
        out_type=jax.ShapeDtypeStruct((seq_len, D_MODEL, batch), jnp.float32),
        scratch_types=(
            [pltpu.VMEM((_BLK,), jnp.int32) for _ in range(_NBUF)]
            + [pltpu.VMEM((_BLK, 128), jnp.float32) for _ in range(_NBUF)]
            + [pltpu.VMEM((D_MODEL, _BLK + 1), jnp.float32) for _ in range(2)]
            + [pltpu.SemaphoreType.DMA for _ in range(_NBUF + 2)]
        ),
        compiler_params=_SC_PARAMS,
    )
    def gather_kernel(xt_hbm, tab_hbm, out_hbm, *scratch):
        idx_bufs = scratch[:_NBUF]
        rows_bufs = scratch[_NBUF : 2 * _NBUF]
        out_bufs = scratch[2 * _NBUF : 2 * _NBUF + 2]
        gsems = scratch[2 * _NBUF + 2 : 3 * _NBUF + 2]
        osems = scratch[3 * _NBUF + 2 : 3 * _NBUF + 4]

        wid = lax.axis_index("s") * info.num_cores + lax.axis_index("c")
        b0 = wid * _BLK

        def issue_gather(s, b):
            pltpu.sync_copy(xt_hbm.at[s, pl.ds(b0, _BLK)], idx_bufs[b])
            pltpu.async_copy(tab_hbm.at[idx_bufs[b]], rows_bufs[b], gsems[b])

        def wait_gather(b):
            pltpu.make_async_copy(
                tab_hbm.at[idx_bufs[b]], rows_bufs[b], gsems[b]
            ).wait()

        def issue_out(s, bo):
            pltpu.async_copy(
                out_bufs[bo].at[:, pl.ds(0, _BLK)],
                out_hbm.at[s, :, pl.ds(b0, _BLK)],
                osems[bo],
            )

        def wait_out(bo):
            pltpu.make_async_copy(
                out_bufs[bo].at[:, pl.ds(0, _BLK)],
                out_hbm.at[0, :, pl.ds(b0, _BLK)],
                osems[bo],
            ).wait()

        def register_pass(b, bo):
            iota = lax.iota(jnp.int32, 16)

            def tbody(t, carry):
                tvec = jnp.full((16,), t, jnp.int32)
                for k in range(D_MODEL // 16):
                    vals = rows_bufs[b][t, pl.ds(k * 16, 16)]
                    plsc.store_scatter(
                        out_bufs[bo], [k * 16 + iota, tvec], vals * _SCALE
                    )
                return carry

            lax.fori_loop(0, _BLK, tbody, 0)

        for i in range(_LOOK):
            issue_gather(i, i)

        def outer(it, carry):
            for b in range(_NBUF):
                i = it * _NBUF + b
                j = i + _LOOK
                bj = (b + _LOOK) % _NBUF
                bo = b % 2

                @pl.when(j < n_chunks)
                def _():
                    issue_gather(j, bj)

                wait_gather(b)

                @pl.when(i >= 2)
                def _():
                    wait_out(bo)

                register_pass(b, bo)
                issue_out(i, bo)
            return carry

        lax.fori_loop(0, n_chunks // _NBUF, outer, 0)

        for bo in range(2):
            wait_out(bo)

    return gather_kernel


def kernel(x, lut):
    batch, seq = x.shape
    vocab = lut.shape[0]
    xt = x.T
    tab = jnp.pad(lut, ((0, 0), (0, 128 - D_MODEL)))
    out_t = _make_gather_kernel(seq, batch, vocab)(xt, tab)
    return out_t.transpose(2, 0, 1)


# speedup vs baseline: 1.5348x; 1.3194x over previous
import functools
import math

import jax
import jax.numpy as jnp
from jax import lax
from jax.experimental import pallas as pl
from jax.experimental.pallas import tpu as pltpu
from jax.experimental.pallas import tpu_sc as plsc

D_MODEL = 64
_SCALE = math.sqrt(D_MODEL)
_BLK = 128
_NBUF = 4
_LOOK = 2

_SC_PARAMS = pltpu.CompilerParams(
    use_tc_tiling_on_sc=True,
    needs_layout_passes=False,
    skip_device_barrier=True,
    disable_semaphore_checks=True,
    disable_bounds_checks=True,
)


@functools.lru_cache(maxsize=None)
def _make_gather_kernel(seq_len: int, batch: int, vocab: int):
    info = plsc.get_sparse_core_info()
    num_workers = info.num_cores * info.num_subcores
    assert batch == num_workers * _BLK
    n_chunks = seq_len
    assert n_chunks % _NBUF == 0 and n_chunks >= 2 * _NBUF

    mesh = plsc.VectorSubcoreMesh(core_axis_name="c", subcore_axis_name="s")

    @functools.partial(
        pl.kernel,
        mesh=mesh,
        out_type=jax.ShapeDtypeStruct((seq_len, D_MODEL, batch), jnp.float32),
        scratch_types=(
            [pltpu.VMEM((_BLK,), jnp.int32) for _ in range(_NBUF)]
            + [pltpu.VMEM((_BLK, 128), jnp.float32) for _ in range(_NBUF)]
            + [pltpu.VMEM((D_MODEL, _BLK + 1), jnp.float32) for _ in range(2)]
            + [pltpu.SemaphoreType.DMA for _ in range(_NBUF + 2)]
        ),
        compiler_params=_SC_PARAMS,
    )
    def gather_kernel(xt_hbm, tab_hbm, out_hbm, *scratch):
        idx_bufs = scratch[:_NBUF]
        rows_bufs = scratch[_NBUF : 2 * _NBUF]
        out_bufs = scratch[2 * _NBUF : 2 * _NBUF + 2]
        gsems = scratch[2 * _NBUF + 2 : 3 * _NBUF + 2]
        osems = scratch[3 * _NBUF + 2 : 3 * _NBUF + 4]

        wid = lax.axis_index("s") * info.num_cores + lax.axis_index("c")
        b0 = wid * _BLK

        def issue_gather(s, b):
            pltpu.sync_copy(xt_hbm.at[s, pl.ds(b0, _BLK)], idx_bufs[b])
            pltpu.async_copy(tab_hbm.at[idx_bufs[b]], rows_bufs[b], gsems[b])

        def wait_gather(b):
            pltpu.make_async_copy(
                tab_hbm.at[idx_bufs[b]], rows_bufs[b], gsems[b]
            ).wait()

        def issue_out(s, bo):
            pltpu.async_copy(
                out_bufs[bo].at[:, pl.ds(0, _BLK)],
                out_hbm.at[s, :, pl.ds(b0, _BLK)],
                osems[bo],
            )

        def wait_out(bo):
            pltpu.make_async_copy(
                out_bufs[bo].at[:, pl.ds(0, _BLK)],
                out_hbm.at[0, :, pl.ds(b0, _BLK)],
                osems[bo],
            ).wait()

        def register_pass(b, bo):
            iota = lax.iota(jnp.int32, 16)

            @plsc.parallel_loop(0, _BLK, unroll=4)
            def tbody(t):
                tvec = jnp.full((16,), t, jnp.int32)
                for k in range(D_MODEL // 16):
                    vals = rows_bufs[b][t, pl.ds(k * 16, 16)]
                    plsc.store_scatter(
                        out_bufs[bo], [k * 16 + iota, tvec], vals * _SCALE
                    )

        for i in range(_LOOK):
            issue_gather(i, i)

        def outer(it, carry):
            for b in range(_NBUF):
                i = it * _NBUF + b
                j = i + _LOOK
                bj = (b + _LOOK) % _NBUF
                bo = b % 2

                @pl.when(j < n_chunks)
                def _():
                    issue_gather(j, bj)

                wait_gather(b)

                @pl.when(i >= 2)
                def _():
                    wait_out(bo)

                register_pass(b, bo)
                issue_out(i, bo)
            return carry

        lax.fori_loop(0, n_chunks // _NBUF, outer, 0)

        for bo in range(2):
            wait_out(bo)

    return gather_kernel


def kernel(x, lut):
    batch, seq = x.shape
    vocab = lut.shape[0]
    xt = x.T
    tab = jnp.pad(lut, ((0, 0), (0, 128 - D_MODEL)))
    out_t = _make_gather_kernel(seq, batch, vocab)(xt, tab)
    return out_t.transpose(2, 0, 1)


# speedup vs baseline: 1.5350x; 1.0002x over previous
import functools
import math

import jax
import jax.numpy as jnp
from jax import lax
from jax.experimental import pallas as pl
from jax.experimental.pallas import tpu as pltpu
from jax.experimental.pallas import tpu_sc as plsc

D_MODEL = 64
_SCALE = math.sqrt(D_MODEL)
_BLK = 128
_NBUF = 4
_LOOK = 2

_SC_PARAMS = pltpu.CompilerParams(
    use_tc_tiling_on_sc=True,
    needs_layout_passes=False,
    skip_device_barrier=True,
    disable_semaphore_checks=True,
    disable_bounds_checks=True,
)


@functools.lru_cache(maxsize=None)
def _make_gather_kernel(seq_len: int, batch: int, vocab: int):
    info = plsc.get_sparse_core_info()
    num_workers = info.num_cores * info.num_subcores
    assert batch == num_workers * _BLK
    n_chunks = seq_len
    assert n_chunks % _NBUF == 0 and n_chunks >= 2 * _NBUF

    mesh = plsc.VectorSubcoreMesh(core_axis_name="c", subcore_axis_name="s")

    @functools.partial(
        pl.kernel,
        mesh=mesh,
        out_type=jax.ShapeDtypeStruct((seq_len, D_MODEL, batch), jnp.float32),
        scratch_types=(
            [pltpu.VMEM((_BLK,), jnp.int32) for _ in range(_NBUF)]
            + [pltpu.VMEM((_BLK, 128), jnp.float32) for _ in range(_NBUF)]
            + [pltpu.VMEM((D_MODEL, _BLK + 1), jnp.float32) for _ in range(2)]
            + [pltpu.SemaphoreType.DMA for _ in range(_NBUF + 2)]
        ),
        compiler_params=_SC_PARAMS,
    )
    def gather_kernel(xt_hbm, tab_hbm, out_hbm, *scratch):
        idx_bufs = scratch[:_NBUF]
        rows_bufs = scratch[_NBUF : 2 * _NBUF]
        out_bufs = scratch[2 * _NBUF : 2 * _NBUF + 2]
        gsems = scratch[2 * _NBUF + 2 : 3 * _NBUF + 2]
        osems = scratch[3 * _NBUF + 2 : 3 * _NBUF + 4]

        wid = lax.axis_index("s") * info.num_cores + lax.axis_index("c")
        b0 = wid * _BLK

        def issue_gather(s, b):
            pltpu.sync_copy(xt_hbm.at[s, pl.ds(b0, _BLK)], idx_bufs[b])
            pltpu.async_copy(tab_hbm.at[idx_bufs[b]], rows_bufs[b], gsems[b])

        def wait_gather(b):
            pltpu.make_async_copy(
                tab_hbm.at[idx_bufs[b]], rows_bufs[b], gsems[b]
            ).wait()

        def issue_out(s, bo):
            pltpu.async_copy(
                out_bufs[bo].at[:, pl.ds(0, _BLK)],
                out_hbm.at[s, :, pl.ds(b0, _BLK)],
                osems[bo],
            )

        def wait_out(bo):
            pltpu.make_async_copy(
                out_bufs[bo].at[:, pl.ds(0, _BLK)],
                out_hbm.at[0, :, pl.ds(b0, _BLK)],
                osems[bo],
            ).wait()

        def register_pass(b, bo):
            iota = lax.iota(jnp.int32, 16)

            @plsc.parallel_loop(0, _BLK, unroll=8)
            def tbody(t):
                tvec = jnp.full((16,), t, jnp.int32)
                for k in range(D_MODEL // 16):
                    vals = rows_bufs[b][t, pl.ds(k * 16, 16)]
                    plsc.store_scatter(
                        out_bufs[bo], [k * 16 + iota, tvec], vals * _SCALE
                    )

        for i in range(_LOOK):
            issue_gather(i, i)

        def outer(it, carry):
            for b in range(_NBUF):
                i = it * _NBUF + b
                j = i + _LOOK
                bj = (b + _LOOK) % _NBUF
                bo = b % 2

                @pl.when(j < n_chunks)
                def _():
                    issue_gather(j, bj)

                wait_gather(b)

                @pl.when(i >= 2)
                def _():
                    wait_out(bo)

                register_pass(b, bo)
                issue_out(i, bo)
            return carry

        lax.fori_loop(0, n_chunks // _NBUF, outer, 0)

        for bo in range(2):
            wait_out(bo)

    return gather_kernel


def kernel(x, lut):
    batch, seq = x.shape
    vocab = lut.shape[0]
    xt = x.T
    tab = jnp.pad(lut, ((0, 0), (0, 128 - D_MODEL)))
    out_t = _make_gather_kernel(seq, batch, vocab)(xt, tab)
    return out_t.transpose(2, 0, 1)


# speedup vs baseline: 2.0465x; 1.3332x over previous
import functools
import math

import jax
import jax.numpy as jnp
from jax import lax
from jax.experimental import pallas as pl
from jax.experimental.pallas import tpu as pltpu
from jax.experimental.pallas import tpu_sc as plsc

D_MODEL = 64
_SCALE = math.sqrt(D_MODEL)
_CHUNK = 160
_NBUF = 4
_LOOK = 2

_SC_PARAMS = pltpu.CompilerParams(
    use_tc_tiling_on_sc=True,
    needs_layout_passes=False,
    skip_device_barrier=True,
    disable_semaphore_checks=True,
    disable_bounds_checks=True,
)


@functools.lru_cache(maxsize=None)
def _make_gather_kernel(n_rows: int):
    info = plsc.get_sparse_core_info()
    num_workers = info.num_cores * info.num_subcores
    rows_per_worker = n_rows // num_workers
    assert rows_per_worker * num_workers == n_rows
    n_chunks = rows_per_worker // _CHUNK
    assert n_chunks * _CHUNK == rows_per_worker
    assert n_chunks % _NBUF == 0 and n_chunks >= 2 * _NBUF

    mesh = plsc.VectorSubcoreMesh(core_axis_name="c", subcore_axis_name="s")

    @functools.partial(
        pl.kernel,
        mesh=mesh,
        out_type=jax.ShapeDtypeStruct((n_rows, D_MODEL), jnp.float32),
        scratch_types=(
            [pltpu.VMEM((_CHUNK,), jnp.int32) for _ in range(_NBUF)]
            + [pltpu.VMEM((_CHUNK, 128), jnp.float32) for _ in range(_NBUF)]
            + [pltpu.VMEM((_CHUNK, D_MODEL), jnp.float32) for _ in range(2)]
            + [pltpu.SemaphoreType.DMA for _ in range(_NBUF + 2)]
        ),
        compiler_params=_SC_PARAMS,
    )
    def gather_kernel(x_hbm, tab_hbm, out_hbm, *scratch):
        idx_bufs = scratch[:_NBUF]
        rows_bufs = scratch[_NBUF : 2 * _NBUF]
        out_bufs = scratch[2 * _NBUF : 2 * _NBUF + 2]
        gsems = scratch[2 * _NBUF + 2 : 3 * _NBUF + 2]
        osems = scratch[3 * _NBUF + 2 : 3 * _NBUF + 4]

        wid = lax.axis_index("s") * info.num_cores + lax.axis_index("c")
        base = wid * rows_per_worker

        def issue_gather(chunk, b):
            off = base + chunk * _CHUNK
            pltpu.sync_copy(x_hbm.at[pl.ds(off, _CHUNK)], idx_bufs[b])
            pltpu.async_copy(tab_hbm.at[idx_bufs[b]], rows_bufs[b], gsems[b])

        def wait_gather(b):
            pltpu.make_async_copy(
                tab_hbm.at[idx_bufs[b]], rows_bufs[b], gsems[b]
            ).wait()

        def issue_out(chunk, bo):
            off = base + chunk * _CHUNK
            pltpu.async_copy(
                out_bufs[bo], out_hbm.at[pl.ds(off, _CHUNK)], osems[bo]
            )

        def wait_out(bo):
            pltpu.make_async_copy(
                out_bufs[bo], out_hbm.at[pl.ds(0, _CHUNK)], osems[bo]
            ).wait()

        def scale_pass(b, bo):
            @plsc.parallel_loop(0, _CHUNK, unroll=8)
            def rbody(r):
                for k in range(D_MODEL // 16):
                    sl = (r, pl.ds(k * 16, 16))
                    out_bufs[bo][sl] = rows_bufs[b][sl] * _SCALE

        for i in range(_LOOK):
            issue_gather(i, i)

        def outer(it, carry):
            for b in range(_NBUF):
                i = it * _NBUF + b
                j = i + _LOOK
                bj = (b + _LOOK) % _NBUF
                bo = b % 2

                @pl.when(j < n_chunks)
                def _():
                    issue_gather(j, bj)

                wait_gather(b)

                @pl.when(i >= 2)
                def _():
                    wait_out(bo)

                scale_pass(b, bo)
                issue_out(i, bo)
            return carry

        lax.fori_loop(0, n_chunks // _NBUF, outer, 0)

        for bo in range(2):
            wait_out(bo)

    return gather_kernel


def kernel(x, lut):
    batch, seq = x.shape
    flat = x.reshape(batch * seq)
    tab = jnp.pad(lut, ((0, 0), (0, 128 - D_MODEL)))
    out = _make_gather_kernel(batch * seq)(flat, tab)
    return out.reshape(batch, seq, D_MODEL)
